# baseline (speedup 1.0000x reference)
import functools

import jax
import jax.numpy as jnp
from jax import lax
from jax.experimental import pallas as pl
from jax.experimental.pallas import tpu as pltpu
from jax.experimental.pallas import tpu_sc as plsc

B, S = 1024, 200
BS = B * S
INTD = 64
GW = 128
HD = 192


_NC, _NS = 2, 16
_NW = _NC * _NS
_PER_W = BS // _NW
_C = 128
_NCHUNK = _PER_W // _C


def _sc_gather_body(idx0, idx1, idx2, idx3, t0, t1, t2, t3,
                    o0, o1, o2, o3, iv0, iv1, iv2, iv3,
                    e0, e1, e2, e3, sem):
    wid = lax.axis_index("s") * _NC + lax.axis_index("c")
    base0 = wid * _PER_W

    def chunk(c, _):
        base = base0 + c * _C
        pltpu.sync_copy(idx0.at[pl.ds(base, _C)], iv0)
        pltpu.sync_copy(idx1.at[pl.ds(base, _C)], iv1)
        pltpu.sync_copy(idx2.at[pl.ds(base, _C)], iv2)
        pltpu.sync_copy(idx3.at[pl.ds(base, _C)], iv3)
        cps = [
            pltpu.async_copy(t0.at[iv0], e0, sem),
            pltpu.async_copy(t1.at[iv1], e1, sem),
            pltpu.async_copy(t2.at[iv2], e2, sem),
            pltpu.async_copy(t3.at[iv3], e3, sem),
        ]
        for cp in cps:
            cp.wait()
        pltpu.sync_copy(e0, o0.at[pl.ds(base, _C)])
        pltpu.sync_copy(e1, o1.at[pl.ds(base, _C)])
        pltpu.sync_copy(e2, o2.at[pl.ds(base, _C)])
        pltpu.sync_copy(e3, o3.at[pl.ds(base, _C)])
        return ()

    lax.fori_loop(0, _NCHUNK, chunk, (), unroll=False)


def _sc_gather(idx0, idx1, idx2, idx3, t0, t1, t2, t3):
    mesh = plsc.VectorSubcoreMesh(core_axis_name="c", subcore_axis_name="s")
    row = jax.ShapeDtypeStruct((BS, GW), jnp.float32)
    f = pl.kernel(
        _sc_gather_body,
        mesh=mesh,
        out_type=(row, row, row, row),
        scratch_types=[
            pltpu.VMEM((_C,), jnp.int32),
            pltpu.VMEM((_C,), jnp.int32),
            pltpu.VMEM((_C,), jnp.int32),
            pltpu.VMEM((_C,), jnp.int32),
            pltpu.VMEM((_C, GW), jnp.float32),
            pltpu.VMEM((_C, GW), jnp.float32),
            pltpu.VMEM((_C, GW), jnp.float32),
            pltpu.VMEM((_C, GW), jnp.float32),
            pltpu.SemaphoreType.DMA,
        ],
    )
    return f(idx0, idx1, idx2, idx3, t0, t1, t2, t3)


_R = 2048


def _tc_body(c0_ref, c1_ref, c2_ref, c3_ref, el_ref, du_ref, w_ref,
             wel_ref, wdu_ref, b_ref, out_ref):
    w = w_ref[...]
    acc = jnp.dot(c0_ref[...], w[0 * GW:1 * GW],
                  preferred_element_type=jnp.float32)
    acc += jnp.dot(c1_ref[...], w[1 * GW:2 * GW],
                   preferred_element_type=jnp.float32)
    acc += jnp.dot(c2_ref[...], w[2 * GW:3 * GW],
                   preferred_element_type=jnp.float32)
    acc += jnp.dot(c3_ref[...], w[3 * GW:4 * GW],
                   preferred_element_type=jnp.float32)
    el = el_ref[...][:, None]
    du = du_ref[...][:, None]
    out_ref[...] = (acc + el * wel_ref[...][None, :] + du * wdu_ref[...][None, :]
                    + b_ref[...][None, :])


def _tc_matmul(c0, c1, c2, c3, el, du, w_top, w_el, w_du, b):
    grid = (BS // _R,)
    row_spec = pl.BlockSpec((_R, GW), lambda i: (i, 0))
    return pl.pallas_call(
        _tc_body,
        grid=grid,
        in_specs=[
            row_spec, row_spec, row_spec, row_spec,
            pl.BlockSpec((_R,), lambda i: (i,)),
            pl.BlockSpec((_R,), lambda i: (i,)),
            pl.BlockSpec((4 * GW, HD), lambda i: (0, 0)),
            pl.BlockSpec((HD,), lambda i: (0,)),
            pl.BlockSpec((HD,), lambda i: (0,)),
            pl.BlockSpec((HD,), lambda i: (0,)),
        ],
        out_specs=pl.BlockSpec((_R, HD), lambda i: (i, 0)),
        out_shape=jax.ShapeDtypeStruct((BS, HD), jnp.float32),
    )(c0, c1, c2, c3, el, du, w_top, w_el, w_du, b)


def kernel(interaction, assessmentItemID, testId, KnowledgeTag, elapsed,
           duration, emb_interaction, emb_assessmentItemID, emb_testId,
           emb_KnowledgeTag, W, b):
    batch_size, seq_len = interaction.shape[0], interaction.shape[1]
    pad = lambda t: jnp.pad(t, ((0, 0), (0, GW - INTD)))
    c0, c1, c2, c3 = _sc_gather(
        interaction.reshape(-1), assessmentItemID.reshape(-1),
        testId.reshape(-1), KnowledgeTag.reshape(-1),
        pad(emb_interaction), pad(emb_assessmentItemID), pad(emb_testId),
        pad(emb_KnowledgeTag))
    w_pad = jnp.concatenate(
        [W[:4 * INTD].reshape(4, INTD, HD),
         jnp.zeros((4, GW - INTD, HD), jnp.float32)], axis=1).reshape(4 * GW, HD)
    X = _tc_matmul(c0, c1, c2, c3, elapsed.reshape(-1), duration.reshape(-1),
                   w_pad, W[4 * INTD], W[4 * INTD + 1], b)
    return (X.reshape(batch_size, seq_len, HD), batch_size, seq_len)
